# baseline (speedup 1.0000x reference)
import jax
import jax.numpy as jnp
from jax import lax
from jax.experimental import pallas as pl
from jax.experimental.pallas import tpu as pltpu

ROW_TILE = 128


def _router_body(h_ref, wr_ref, wab_ref, slots_ref, te_ref):
    h = h_ref[...]
    logits = jnp.dot(h, wr_ref[...], preferred_element_type=jnp.float32)
    s, e = logits.shape
    n_tiles = te_ref.shape[0]
    col = lax.broadcasted_iota(jnp.int32, (s, e), 1)
    a1 = jnp.argmax(logits, axis=1).astype(jnp.int32)
    m1 = jnp.max(logits, axis=1)
    masked = jnp.where(col == a1[:, None], -jnp.inf, logits)
    a2 = jnp.argmax(masked, axis=1).astype(jnp.int32)
    m2 = jnp.max(masked, axis=1)
    t = jnp.exp(m2 - m1)
    wa = 1.0 / (1.0 + t)
    wb = 1.0 - wa
    wab_ref[...] = jnp.concatenate([wa[:, None], wb[:, None]], axis=1)

    oh1 = (col == a1[:, None]).astype(jnp.float32)
    oh2 = (col == a2[:, None]).astype(jnp.float32)
    st = oh1 + oh2
    ri = lax.broadcasted_iota(jnp.int32, (ROW_TILE, ROW_TILE), 0)
    ci = lax.broadcasted_iota(jnp.int32, (ROW_TILE, ROW_TILE), 1)
    lt = (ci < ri).astype(jnp.float32)
    off = jnp.zeros((1, e), jnp.float32)
    parts = []
    for b in range(s // ROW_TILE):
        blk = st[b * ROW_TILE:(b + 1) * ROW_TILE, :]
        pin = jnp.dot(lt, blk, preferred_element_type=jnp.float32)
        parts.append(pin + off)
        off = off + jnp.sum(blk, axis=0, keepdims=True)
    p = jnp.concatenate(parts, axis=0)
    counts = off
    rank1 = jnp.sum(p * oh1, axis=1)
    rank2 = jnp.sum(p * oh2, axis=1)
    tiles_per = jnp.floor((counts + (ROW_TILE - 1)) * (1.0 / ROW_TILE))
    r16 = lax.broadcasted_iota(jnp.int32, (e, e), 0)
    c16 = lax.broadcasted_iota(jnp.int32, (e, e), 1)
    m16 = (r16 < c16).astype(jnp.float32)
    ts = jnp.dot(tiles_per, m16, preferred_element_type=jnp.float32)
    pstart = ts * float(ROW_TILE)
    slot1 = jnp.sum(oh1 * pstart, axis=1) + rank1
    slot2 = jnp.sum(oh2 * pstart, axis=1) + rank2
    slots_ref[...] = jnp.concatenate(
        [slot1[:, None], slot2[:, None]], axis=1).astype(jnp.int32)
    tid = lax.broadcasted_iota(jnp.int32, (n_tiles, e), 0)
    ts_i = ts.astype(jnp.int32)
    ge = (tid >= ts_i).astype(jnp.int32)
    te = jnp.sum(ge, axis=1) - 1
    oh_last = (lax.broadcasted_iota(jnp.int32, (1, e), 1) == (e - 1))
    total = jnp.sum(jnp.where(oh_last, ts + tiles_per, 0.0), axis=1,
                    keepdims=True).astype(jnp.int32)
    te = jnp.where(tid[:, 0:1] < total, te[:, None], -1)
    te_ref[...] = te


def kernel(hidden_states, w_router, w1, v1, w2):
    batch, seq, d_model = hidden_states.shape
    n_experts, _, ffn = w1.shape
    s = batch * seq
    top_k = 2
    n_assign = s * top_k
    n_tiles = n_assign // ROW_TILE + n_experts - 1
    n_pad = n_tiles * ROW_TILE
    n_tok_tiles = s // ROW_TILE

    h2 = hidden_states.reshape(s, d_model)

    wab, slots, te2 = pl.pallas_call(
        _router_body,
        out_shape=(
            jax.ShapeDtypeStruct((s, top_k), jnp.float32),
            jax.ShapeDtypeStruct((s, top_k), jnp.int32),
            jax.ShapeDtypeStruct((n_tiles, 1), jnp.int32),
        ),
    )(h2, w_router)

    token_map = jnp.zeros((n_pad,), jnp.int32).at[slots.reshape(-1)].set(
        jnp.arange(n_assign, dtype=jnp.int32) // top_k,
        mode="promise_in_bounds", unique_indices=True)
    tile_expert = te2.reshape(-1)
    pa, pb = slots[:, 0], slots[:, 1]

    def _ffn_body(te_ref, tm_ref, h_ref, w1_ref, v1_ref, w2_ref, y_ref,
                  x_scr):
        i = pl.program_id(0)
        expert = te_ref[i]

        @pl.when(expert >= 0)
        def _():
            def gather_row(r, carry):
                tok = tm_ref[i * ROW_TILE + r]
                x_scr[pl.ds(r, 1), :] = h_ref[pl.ds(tok, 1), :]
                return carry

            lax.fori_loop(0, ROW_TILE, gather_row, 0)
            x = x_scr[...]
            t1 = jnp.dot(x, w1_ref[0], preferred_element_type=jnp.float32)
            t2 = jnp.dot(x, v1_ref[0], preferred_element_type=jnp.float32)
            g = t1 * jax.nn.sigmoid(t1) * t2
            y_ref[...] = jnp.dot(g, w2_ref[0],
                                 preferred_element_type=jnp.float32)

    grid_spec = pltpu.PrefetchScalarGridSpec(
        num_scalar_prefetch=2,
        grid=(n_tiles,),
        in_specs=[
            pl.BlockSpec((s, d_model), lambda i, te_, tm: (0, 0)),
            pl.BlockSpec((1, d_model, ffn),
                         lambda i, te_, tm: (jnp.maximum(te_[i], 0), 0, 0)),
            pl.BlockSpec((1, d_model, ffn),
                         lambda i, te_, tm: (jnp.maximum(te_[i], 0), 0, 0)),
            pl.BlockSpec((1, ffn, d_model),
                         lambda i, te_, tm: (jnp.maximum(te_[i], 0), 0, 0)),
        ],
        out_specs=pl.BlockSpec((ROW_TILE, d_model), lambda i, te_, tm: (i, 0)),
        scratch_shapes=[pltpu.VMEM((ROW_TILE, d_model), jnp.float32)],
    )
    y = pl.pallas_call(
        _ffn_body,
        grid_spec=grid_spec,
        out_shape=jax.ShapeDtypeStruct((n_pad, d_model), jnp.float32),
        compiler_params=pltpu.CompilerParams(
            vmem_limit_bytes=100 * 1024 * 1024),
    )(tile_expert, token_map, h2, w1, v1, w2)

    def _combine_body(pa_ref, pb_ref, wab_ref, y_ref, out_ref, ya_scr,
                      yb_scr):
        i = pl.program_id(0)

        def gather_row(r, carry):
            pa_ = pa_ref[i * ROW_TILE + r]
            pb_ = pb_ref[i * ROW_TILE + r]
            ya_scr[pl.ds(r, 1), :] = y_ref[pl.ds(pa_, 1), :]
            yb_scr[pl.ds(r, 1), :] = y_ref[pl.ds(pb_, 1), :]
            return carry

        lax.fori_loop(0, ROW_TILE, gather_row, 0)
        wa = wab_ref[:, 0:1]
        wb = wab_ref[:, 1:2]
        out_ref[...] = wa * ya_scr[...] + wb * yb_scr[...]

    comb_spec = pltpu.PrefetchScalarGridSpec(
        num_scalar_prefetch=2,
        grid=(n_tok_tiles,),
        in_specs=[
            pl.BlockSpec((ROW_TILE, top_k), lambda i, pa_, pb_: (i, 0)),
            pl.BlockSpec((n_pad, d_model), lambda i, pa_, pb_: (0, 0)),
        ],
        out_specs=pl.BlockSpec((ROW_TILE, d_model), lambda i, pa_, pb_: (i, 0)),
        scratch_shapes=[
            pltpu.VMEM((ROW_TILE, d_model), jnp.float32),
            pltpu.VMEM((ROW_TILE, d_model), jnp.float32),
        ],
    )
    out = pl.pallas_call(
        _combine_body,
        grid_spec=comb_spec,
        out_shape=jax.ShapeDtypeStruct((s, d_model), jnp.float32),
        compiler_params=pltpu.CompilerParams(
            vmem_limit_bytes=100 * 1024 * 1024),
    )(pa, pb, wab, y)

    return out.reshape(batch, seq, d_model)
